# Initial kernel scaffold; baseline (speedup 1.0000x reference)
import jax
import jax.numpy as jnp
from jax import lax
from jax.experimental import pallas as pl
from jax.experimental.pallas import tpu as pltpu
from jax.experimental.pallas import tpu_sc as plsc

N = 10000
E = 320000
HALF = E // 2
DN = 128
DE = 16
H = 64
G = 256
GS = 32
T = 12
R1O = GS + H + 50
R2O = int(2 / 3 * R1O) + T

NC = 2
NS = 16
NW = NC * NS
EW = E // NW
CH = 80
NCH = EW // CH
NROW = N // NS

_F32 = jnp.float32


def _dot(a, b):
    return lax.dot_general(a, b, (((1,), (1,)), ((), ())),
                           preferred_element_type=_F32)


def _sc_gather_body(table, idx, out, idx_v, rows, gsem, ssem):
    wid = lax.axis_index("s") * NC + lax.axis_index("c")
    base = wid * EW
    pltpu.sync_copy(idx.at[wid], idx_v)
    NBUF = 8
    gdesc = [None] * NCH
    sdesc = [None] * NCH

    def fire_gather(c):
        gdesc[c] = pltpu.async_copy(table.at[idx_v.at[c]], rows.at[c % NBUF],
                                    gsem)

    def fire_store(c):
        sdesc[c] = pltpu.async_copy(
            rows.at[c % NBUF], out.at[pl.ds(base + c * CH, CH)], ssem)

    for c in range(min(NBUF, NCH)):
        fire_gather(c)
    for c in range(NCH):
        gdesc[c].wait()
        fire_store(c)
        if c >= NBUF - 1:
            sdesc[c - NBUF + 1].wait()
        if c + NBUF < NCH:
            fire_gather(c + NBUF)
    for c in range(max(0, NCH - NBUF + 1), NCH):
        sdesc[c].wait()


_sc_gather = pl.kernel(
    _sc_gather_body,
    out_type=jax.ShapeDtypeStruct((E, H), _F32),
    mesh=plsc.VectorSubcoreMesh(core_axis_name="c", subcore_axis_name="s"),
    scratch_types=[
        pltpu.VMEM((NCH, CH), jnp.int32),
        pltpu.VMEM((8, CH, H), _F32),
        pltpu.SemaphoreType.DMA,
        pltpu.SemaphoreType.DMA,
    ],
)


def _sc_scatter_body(rows, idx, zeros, out, idx_v, hbuf, table_sh, lsem):
    cid = lax.axis_index("c")
    sid = lax.axis_index("s")
    wid = sid * NC + cid
    base = wid * EW
    pltpu.sync_copy(zeros.at[pl.ds(sid * NROW, NROW)],
                    table_sh.at[pl.ds(sid * NROW, NROW)])
    pltpu.sync_copy(idx.at[wid], idx_v)
    plsc.subcore_barrier()

    NBUF = 4
    ldesc = [None] * NCH

    def fire_load(c):
        ldesc[c] = pltpu.async_copy(
            rows.at[pl.ds(base + c * CH, CH)], hbuf.at[c % NBUF], lsem)

    for c in range(min(NBUF, NCH)):
        fire_load(c)
    for c in range(NCH):
        ldesc[c].wait()
        pltpu.sync_copy(hbuf.at[c % NBUF], table_sh.at[idx_v.at[c]], add=True)
        if c + NBUF < NCH:
            fire_load(c + NBUF)
    plsc.subcore_barrier()
    pltpu.sync_copy(table_sh.at[pl.ds(sid * NROW, NROW)],
                    out.at[cid, pl.ds(sid * NROW, NROW)])


_sc_scatter = pl.kernel(
    _sc_scatter_body,
    out_type=jax.ShapeDtypeStruct((NC, N, H), _F32),
    mesh=plsc.VectorSubcoreMesh(core_axis_name="c", subcore_axis_name="s"),
    scratch_types=[
        pltpu.VMEM((NCH, CH), jnp.int32),
        pltpu.VMEM((4, CH, H), _F32),
        pltpu.VMEM_SHARED((N, H), _F32),
        pltpu.SemaphoreType.DMA,
    ],
)


def _node_proj_body(x_ref, w_ref, o_ref):
    o_ref[...] = _dot(x_ref[...], w_ref[...])


def _h0_body(pv_ref, e_ref, wie_ref, bi_ref, o_ref):
    o_ref[...] = jax.nn.relu(pv_ref[...] + _dot(e_ref[...], wie_ref[...])
                             + bi_ref[...])


def _merge_matmul_body(parts_ref, w_ref, o_ref):
    o_ref[...] = _dot(parts_ref[0] + parts_ref[1], w_ref[...])


def _combine1_body(h0_ref, bv_ref, wm_ref, bm_ref, o_ref):
    c_lo = _dot(h0_ref[0], wm_ref[...])
    c_hi = _dot(h0_ref[1], wm_ref[...])
    o_ref[0] = jax.nn.relu(h0_ref[0] + bv_ref[0] - c_hi + bm_ref[...])
    o_ref[1] = jax.nn.relu(h0_ref[1] + bv_ref[1] - c_lo + bm_ref[...])


def _combine2_body(h0_ref, h_ref, bv_ref, wm_ref, bm_ref, o_ref):
    c_lo = _dot(h_ref[0], wm_ref[...])
    c_hi = _dot(h_ref[1], wm_ref[...])
    o_ref[0] = jax.nn.relu(h0_ref[0] + bv_ref[0] - c_hi + bm_ref[...])
    o_ref[1] = jax.nn.relu(h0_ref[1] + bv_ref[1] - c_lo + bm_ref[...])


def _final_body(x_ref, parts_ref, gf_ref, gid_ref, wav_ref, wam_ref, ba_ref,
                w1v_ref, w1g_ref, b1_ref, w2_ref, b2_ref, w3_ref, b3_ref,
                o_ref):
    m_v = parts_ref[0] + parts_ref[1]
    h_v = jax.nn.relu(_dot(x_ref[...], wav_ref[...]) + _dot(m_v, wam_ref[...])
                      + ba_ref[...])
    giota = lax.broadcasted_iota(jnp.int32, (G, N), 0)
    mask = (giota == gid_ref[...]).astype(_F32)
    sums = lax.dot_general(mask, h_v, (((1,), (0,)), ((), ())),
                           preferred_element_type=_F32)
    counts = jnp.sum(mask, axis=1, keepdims=True)
    mean = sums / jnp.maximum(counts, 1.0)
    r1 = jax.nn.relu(_dot(mean, w1v_ref[...]) + _dot(gf_ref[...], w1g_ref[...])
                     + b1_ref[...])
    r2 = jax.nn.relu(_dot(r1, w2_ref[...]) + b2_ref[...])
    o_ref[...] = _dot(r2, w3_ref[...]) + b3_ref[...]


BE = 4000
BH = 4000


def _h0_call(pv, ea, wie_p, bi2):
    return pl.pallas_call(
        _h0_body,
        grid=(E // BE,),
        in_specs=[
            pl.BlockSpec((BE, H), lambda i: (i, 0)),
            pl.BlockSpec((BE, DE + 1), lambda i: (i, 0)),
            pl.BlockSpec((H, DE + 1), lambda i: (0, 0)),
            pl.BlockSpec((1, H), lambda i: (0, 0)),
        ],
        out_specs=pl.BlockSpec((BE, H), lambda i: (i, 0)),
        out_shape=jax.ShapeDtypeStruct((E, H), _F32),
    )(pv, ea, wie_p, bi2)


def _combine_call(body, args):
    n_big = len(args) - 2
    big = pl.BlockSpec((2, BH, H), lambda i: (0, i, 0))
    return pl.pallas_call(
        body,
        grid=(HALF // BH,),
        in_specs=[big] * n_big + [
            pl.BlockSpec((H, H), lambda i: (0, 0)),
            pl.BlockSpec((1, H), lambda i: (0, 0)),
        ],
        out_specs=big,
        out_shape=jax.ShapeDtypeStruct((2, HALF, H), _F32),
    )(*args)


def kernel(x, edge_attr, g_feat, edge_index, graph_ids, Wi, bi, Wm, bm,
           Wa, ba, W1, b1, W2, b2, W3, b3):
    f32 = _F32
    src = edge_index[0].astype(jnp.int32)
    dst = edge_index[1].astype(jnp.int32)
    src_w = src.reshape(NW, NCH, CH)
    dst_w = dst.reshape(NW, NCH, CH)
    zeros_tab = jnp.zeros((N, H), f32)

    zcol = jnp.zeros((H, 1), f32)
    wiv_p = jnp.concatenate([Wi[:, :DN], zcol], axis=1)
    wie_p = jnp.concatenate([Wi[:, DN:], zcol], axis=1)
    wav_p = jnp.concatenate([Wa[:, :DN], zcol], axis=1)
    wam = Wa[:, DN:]
    w1v = W1[:, :H]
    w1g = W1[:, H:]
    bi2 = bi.reshape(1, H)
    bm2 = bm.reshape(1, H)
    ba2 = ba.reshape(1, H)
    b12 = b1.reshape(1, R1O)
    b22 = b2.reshape(1, R2O)
    b32 = b3.reshape(1, T)
    gid2 = graph_ids.astype(jnp.int32).reshape(1, N)

    P = pl.pallas_call(
        _node_proj_body,
        out_shape=jax.ShapeDtypeStruct((N, H), f32),
    )(x, wiv_p)

    pv = _sc_gather(P, src_w)
    h0 = _h0_call(pv, edge_attr, wie_p, bi2)
    h0r = h0.reshape(2, HALF, H)

    hr = h0r
    for layer in range(2):
        h = hr.reshape(E, H)
        parts = _sc_scatter(h, dst_w, zeros_tab)
        B = pl.pallas_call(
            _merge_matmul_body,
            out_shape=jax.ShapeDtypeStruct((N, H), f32),
        )(parts, Wm)
        bv = _sc_gather(B, src_w).reshape(2, HALF, H)
        if layer == 0:
            hr = _combine_call(_combine1_body, (h0r, bv, Wm, bm2))
        else:
            hr = _combine_call(_combine2_body, (h0r, hr, bv, Wm, bm2))

    parts_v = _sc_scatter(hr.reshape(E, H), src_w, zeros_tab)

    out = pl.pallas_call(
        _final_body,
        out_shape=jax.ShapeDtypeStruct((G, T), f32),
    )(x, parts_v, g_feat, gid2, wav_p, wam, ba2,
      w1v, w1g, b12, W2, b22, W3, b32)
    return out


# speedup vs baseline: 2.4897x; 2.4897x over previous
import functools

import jax
import jax.numpy as jnp
from jax import lax
from jax.experimental import pallas as pl
from jax.experimental.pallas import tpu as pltpu
from jax.experimental.pallas import tpu_sc as plsc

N = 10000
E = 320000
HALF = E // 2
DN = 128
DE = 16
H = 64
G = 256
GS = 32
T = 12
R1O = GS + H + 50
R2O = int(2 / 3 * R1O) + T

NC = 2
NS = 16
NW = NC * NS
EW = E // NW
CH = 80
NCH = EW // CH
NROW = N // NS

_F32 = jnp.float32


def _dot(a, b):
    return lax.dot_general(a, b, (((1,), (1,)), ((), ())),
                           precision=lax.Precision.HIGHEST,
                           preferred_element_type=_F32)


KG = 5
NG = NCH // KG
GR = KG * CH


def _sc_gather_body(table, idx, out, idx_v, rows, gsem, ssem):
    wid = lax.axis_index("s") * NC + lax.axis_index("c")
    base = wid * EW
    pltpu.sync_copy(idx.at[wid], idx_v)
    gdescs = [[None] * KG for _ in range(NG)]
    sdesc = [None] * NG

    def fire_gathers(g):
        p = g % 2
        for j in range(KG):
            c = g * KG + j
            gdescs[g][j] = pltpu.async_copy(
                table.at[idx_v.at[c]], rows.at[p, pl.ds(j * CH, CH)], gsem)

    def fire_store(g):
        sdesc[g] = pltpu.async_copy(
            rows.at[g % 2], out.at[pl.ds(base + g * GR, GR)], ssem)

    fire_gathers(0)
    for g in range(NG):
        for d in gdescs[g]:
            d.wait()
        fire_store(g)
        if g + 1 < NG:
            if g >= 1:
                sdesc[g - 1].wait()
            fire_gathers(g + 1)
    for g in (NG - 2, NG - 1):
        sdesc[g].wait()


@functools.cache
def _sc_gather():
    return pl.kernel(
        _sc_gather_body,
        out_type=jax.ShapeDtypeStruct((E, H), _F32),
        mesh=plsc.VectorSubcoreMesh(core_axis_name="c", subcore_axis_name="s",
                                    num_cores=NC, num_subcores=NS),
        scratch_types=[
            pltpu.VMEM((NCH, CH), jnp.int32),
            pltpu.VMEM((2, GR, H), _F32),
            pltpu.SemaphoreType.DMA,
            pltpu.SemaphoreType.DMA,
        ],
        compiler_params=pltpu.CompilerParams(use_tc_tiling_on_sc=False),
    )


def _sc_scatter_body(rows, idx, zeros, out, idx_v, hbuf, table_sh, lsem, wsem):
    cid = lax.axis_index("c")
    sid = lax.axis_index("s")
    wid = sid * NC + cid
    base = wid * EW
    pltpu.sync_copy(zeros.at[pl.ds(sid * NROW, NROW)],
                    table_sh.at[pl.ds(sid * NROW, NROW)])
    pltpu.sync_copy(idx.at[wid], idx_v)
    plsc.subcore_barrier()

    ldesc = [None] * NG
    wdescs = [[None] * KG for _ in range(NG)]

    def fire_load(g):
        ldesc[g] = pltpu.async_copy(
            rows.at[pl.ds(base + g * GR, GR)], hbuf.at[g % 2], lsem)

    def fire_scatters(g):
        p = g % 2
        for j in range(KG):
            c = g * KG + j
            wdescs[g][j] = pltpu.async_copy(
                hbuf.at[p, pl.ds(j * CH, CH)], table_sh.at[idx_v.at[c]],
                wsem, add=True)

    fire_load(0)
    for g in range(NG):
        ldesc[g].wait()
        if g + 1 < NG:
            if g >= 1:
                for d in wdescs[g - 1]:
                    d.wait()
            fire_load(g + 1)
        fire_scatters(g)
    for g in (NG - 2, NG - 1):
        for d in wdescs[g]:
            d.wait()
    plsc.subcore_barrier()
    pltpu.sync_copy(table_sh.at[pl.ds(sid * NROW, NROW)],
                    out.at[cid, pl.ds(sid * NROW, NROW)])


@functools.cache
def _sc_scatter():
    return pl.kernel(
        _sc_scatter_body,
        out_type=jax.ShapeDtypeStruct((NC, N, H), _F32),
        mesh=plsc.VectorSubcoreMesh(core_axis_name="c", subcore_axis_name="s",
                                    num_cores=NC, num_subcores=NS),
        scratch_types=[
            pltpu.VMEM((NCH, CH), jnp.int32),
            pltpu.VMEM((2, GR, H), _F32),
            pltpu.VMEM_SHARED((N, H), _F32),
            pltpu.SemaphoreType.DMA,
            pltpu.SemaphoreType.DMA,
        ],
        compiler_params=pltpu.CompilerParams(use_tc_tiling_on_sc=False),
    )


def _node_proj_body(x_ref, w_ref, o_ref):
    o_ref[...] = _dot(x_ref[...], w_ref[...])


def _h0_body(pv_ref, e_ref, wie_ref, bi_ref, o_ref):
    o_ref[...] = jax.nn.relu(pv_ref[...] + _dot(e_ref[...], wie_ref[...])
                             + bi_ref[...])


def _merge_matmul_body(parts_ref, w_ref, o_ref):
    o_ref[...] = _dot(parts_ref[0] + parts_ref[1], w_ref[...])


def _combine1_body(h0_ref, bv_ref, wm_ref, bm_ref, o_ref):
    c_lo = _dot(h0_ref[0], wm_ref[...])
    c_hi = _dot(h0_ref[1], wm_ref[...])
    o_ref[0] = jax.nn.relu(h0_ref[0] + bv_ref[0] - c_hi + bm_ref[...])
    o_ref[1] = jax.nn.relu(h0_ref[1] + bv_ref[1] - c_lo + bm_ref[...])


def _combine2_body(h0_ref, h_ref, bv_ref, wm_ref, bm_ref, o_ref):
    c_lo = _dot(h_ref[0], wm_ref[...])
    c_hi = _dot(h_ref[1], wm_ref[...])
    o_ref[0] = jax.nn.relu(h0_ref[0] + bv_ref[0] - c_hi + bm_ref[...])
    o_ref[1] = jax.nn.relu(h0_ref[1] + bv_ref[1] - c_lo + bm_ref[...])


def _final_body(x_ref, parts_ref, gf_ref, gid_ref, wav_ref, wam_ref, ba_ref,
                w1v_ref, w1g_ref, b1_ref, w2_ref, b2_ref, w3_ref, b3_ref,
                o_ref):
    m_v = parts_ref[0] + parts_ref[1]
    h_v = jax.nn.relu(_dot(x_ref[...], wav_ref[...]) + _dot(m_v, wam_ref[...])
                      + ba_ref[...])
    giota = lax.broadcasted_iota(jnp.int32, (G, N), 0)
    mask = (giota == gid_ref[...]).astype(_F32)
    sums = lax.dot_general(mask, h_v, (((1,), (0,)), ((), ())),
                           precision=lax.Precision.HIGHEST,
                           preferred_element_type=_F32)
    counts = jnp.sum(mask, axis=1, keepdims=True)
    mean = sums / jnp.maximum(counts, 1.0)
    r1 = jax.nn.relu(_dot(mean, w1v_ref[...]) + _dot(gf_ref[...], w1g_ref[...])
                     + b1_ref[...])
    r2 = jax.nn.relu(_dot(r1, w2_ref[...]) + b2_ref[...])
    o_ref[...] = _dot(r2, w3_ref[...]) + b3_ref[...]


BE = 4000
BH = 4000


def _h0_call(pv, ea, wie_p, bi2):
    return pl.pallas_call(
        _h0_body,
        grid=(E // BE,),
        in_specs=[
            pl.BlockSpec((BE, H), lambda i: (i, 0)),
            pl.BlockSpec((BE, DE + 1), lambda i: (i, 0)),
            pl.BlockSpec((H, DE + 1), lambda i: (0, 0)),
            pl.BlockSpec((1, H), lambda i: (0, 0)),
        ],
        out_specs=pl.BlockSpec((BE, H), lambda i: (i, 0)),
        out_shape=jax.ShapeDtypeStruct((E, H), _F32),
    )(pv, ea, wie_p, bi2)


def _combine_call(body, args):
    n_big = len(args) - 2
    big = pl.BlockSpec((2, BH, H), lambda i: (0, i, 0))
    return pl.pallas_call(
        body,
        grid=(HALF // BH,),
        in_specs=[big] * n_big + [
            pl.BlockSpec((H, H), lambda i: (0, 0)),
            pl.BlockSpec((1, H), lambda i: (0, 0)),
        ],
        out_specs=big,
        out_shape=jax.ShapeDtypeStruct((2, HALF, H), _F32),
    )(*args)


def kernel(x, edge_attr, g_feat, edge_index, graph_ids, Wi, bi, Wm, bm,
           Wa, ba, W1, b1, W2, b2, W3, b3):
    f32 = _F32
    src = edge_index[0].astype(jnp.int32)
    dst = edge_index[1].astype(jnp.int32)
    src_w = src.reshape(NW, NCH, CH)
    dst_w = dst.reshape(NW, NCH, CH)
    zeros_tab = jnp.zeros((N, H), f32)

    zcol = jnp.zeros((H, 1), f32)
    wiv_p = jnp.concatenate([Wi[:, :DN], zcol], axis=1)
    wie_p = jnp.concatenate([Wi[:, DN:], zcol], axis=1)
    wav_p = jnp.concatenate([Wa[:, :DN], zcol], axis=1)
    wam = Wa[:, DN:]
    w1v = W1[:, :H]
    w1g = W1[:, H:]
    bi2 = bi.reshape(1, H)
    bm2 = bm.reshape(1, H)
    ba2 = ba.reshape(1, H)
    b12 = b1.reshape(1, R1O)
    b22 = b2.reshape(1, R2O)
    b32 = b3.reshape(1, T)
    gid2 = graph_ids.astype(jnp.int32).reshape(1, N)

    P = pl.pallas_call(
        _node_proj_body,
        out_shape=jax.ShapeDtypeStruct((N, H), f32),
    )(x, wiv_p)

    pv = _sc_gather()(P, src_w)
    h0 = _h0_call(pv, edge_attr, wie_p, bi2)
    h0r = h0.reshape(2, HALF, H)

    hr = h0r
    for layer in range(2):
        h = hr.reshape(E, H)
        parts = _sc_scatter()(h, dst_w, zeros_tab)
        B = pl.pallas_call(
            _merge_matmul_body,
            out_shape=jax.ShapeDtypeStruct((N, H), f32),
        )(parts, Wm)
        bv = _sc_gather()(B, src_w).reshape(2, HALF, H)
        if layer == 0:
            hr = _combine_call(_combine1_body, (h0r, bv, Wm, bm2))
        else:
            hr = _combine_call(_combine2_body, (h0r, hr, bv, Wm, bm2))

    parts_v = _sc_scatter()(hr.reshape(E, H), src_w, zeros_tab)

    out = pl.pallas_call(
        _final_body,
        out_shape=jax.ShapeDtypeStruct((G, T), f32),
    )(x, parts_v, g_feat, gid2, wav_p, wam, ba2,
      w1v, w1g, b12, W2, b22, W3, b32)
    return out
